# baseline (speedup 1.0000x reference)
import functools

import jax
import jax.numpy as jnp
from jax import lax
from jax.experimental import pallas as pl
from jax.experimental.pallas import tpu as pltpu
from jax.experimental.pallas import tpu_sc as plsc

_L = 16


def kernel(user_ids, table):
    (B,) = user_ids.shape
    V, D = table.shape
    pack = 128 // D
    info = plsc.get_sparse_core_info()
    nw = info.num_cores * info.num_subcores
    b_per_w = B // nw
    n_grp = b_per_w // _L

    table_w = table.reshape(V // pack, 128)
    mesh = plsc.VectorSubcoreMesh(core_axis_name="c", subcore_axis_name="s")

    @functools.partial(
        pl.kernel,
        mesh=mesh,
        out_type=jax.ShapeDtypeStruct((B // pack, 128), jnp.float32),
        compiler_params=pltpu.CompilerParams(needs_layout_passes=False),
        scratch_types=[
            pltpu.VMEM((b_per_w,), jnp.int32),
            pltpu.VMEM((b_per_w,), jnp.int32),
            pltpu.VMEM((b_per_w, 128), jnp.float32),
            pltpu.VMEM((b_per_w // pack, 128), jnp.float32),
            pltpu.SemaphoreType.DMA,
        ],
    )
    def gather_kernel(idx_hbm, table_hbm, out_hbm, idx_v, line_v, rows_v,
                      out_v, sem):
        wid = lax.axis_index("s") * info.num_cores + lax.axis_index("c")
        base = pl.multiple_of(wid * b_per_w, b_per_w)
        pltpu.sync_copy(idx_hbm.at[pl.ds(base, b_per_w)], idx_v)
        for g in range(n_grp):
            ids = idx_v[pl.ds(g * _L, _L)]
            line_v[pl.ds(g * _L, _L)] = lax.shift_right_logical(ids, 2)
        copy = pltpu.async_copy(table_hbm.at[line_v], rows_v, sem)

        lane = lax.iota(jnp.int32, _L)
        copy.wait()

        def extract(g, carry):
            rows16 = g * _L + lane
            ids16 = plsc.load_gather(idx_v, [rows16])
            src_col = lax.shift_left(jnp.bitwise_and(ids16, pack - 1),
                                     5)
            dst_row = lax.shift_right_logical(rows16, 2)
            dst_col = lax.shift_left(jnp.bitwise_and(rows16, pack - 1), 5)
            for c in range(D):
                val = plsc.load_gather(rows_v, [rows16, src_col + c])
                plsc.store_scatter(out_v, [dst_row, dst_col + c], val)
            return carry

        lax.fori_loop(0, n_grp, extract, 0)
        obase = pl.multiple_of(wid * (b_per_w // pack), b_per_w // pack)
        pltpu.sync_copy(out_v, out_hbm.at[pl.ds(obase, b_per_w // pack)])

    out_w = gather_kernel(user_ids.astype(jnp.int32), table_w)
    return out_w.reshape(B, D)


# speedup vs baseline: 1.0234x; 1.0234x over previous
import functools

import jax
import jax.numpy as jnp
from jax import lax
from jax.experimental import pallas as pl
from jax.experimental.pallas import tpu as pltpu
from jax.experimental.pallas import tpu_sc as plsc

_PACK = 4
_CHUNK = 128
_LANES = 16


def kernel(user_ids, table):
    (B,) = user_ids.shape
    V, D = table.shape
    info = plsc.get_sparse_core_info()
    nw = info.num_cores * info.num_subcores
    b_per_w = B // nw
    n_chunks = b_per_w // _CHUNK
    dp = D * _PACK
    out_rows_w = b_per_w // _PACK
    segs = dp // _LANES
    h_per_id = D // _LANES

    idx = user_ids.astype(jnp.int32)
    pidx3 = (idx // _PACK).reshape(nw, n_chunks, _CHUNK)
    rem3 = jnp.repeat(idx % _PACK, D).reshape(nw, out_rows_w, dp)
    table_p = table.reshape(V // _PACK, dp)

    mesh = plsc.VectorSubcoreMesh(core_axis_name="c", subcore_axis_name="s")

    @functools.partial(
        pl.kernel,
        mesh=mesh,
        out_type=jax.ShapeDtypeStruct((B // _PACK, dp), jnp.float32),
        scratch_types=[
            pltpu.VMEM((n_chunks, _CHUNK), jnp.int32),
            pltpu.VMEM((out_rows_w, dp), jnp.int32),
            pltpu.VMEM((b_per_w, dp), jnp.float32),
            pltpu.VMEM((out_rows_w, dp), jnp.float32),
            pltpu.SemaphoreType.DMA,
        ],
    )
    def gather_kernel(
        pidx_hbm, rem_hbm, table_hbm, out_hbm, pidx_v, rem_v, rows_v, out_v, sem
    ):
        wid = lax.axis_index("s") * info.num_cores + lax.axis_index("c")
        pltpu.sync_copy(pidx_hbm.at[wid], pidx_v)
        pltpu.sync_copy(rem_hbm.at[wid], rem_v)
        copies = [
            pltpu.async_copy(
                table_hbm.at[pidx_v.at[j]],
                rows_v.at[pl.ds(j * _CHUNK, _CHUNK)],
                sem,
            )
            for j in range(n_chunks)
        ]
        for c in copies:
            c.wait()

        def body(o, carry):
            for seg in range(segs):
                j, h = divmod(seg, h_per_id)
                src = o * _PACK + j
                rv = rem_v[o, pl.ds(seg * _LANES, _LANES)]
                val = rows_v[src, pl.ds((_PACK - 1) * D + h * _LANES, _LANES)]
                for r in range(_PACK - 2, -1, -1):
                    cand = rows_v[src, pl.ds(r * D + h * _LANES, _LANES)]
                    val = jnp.where(rv == r, cand, val)
                out_v[o, pl.ds(seg * _LANES, _LANES)] = val
            return carry

        lax.fori_loop(0, out_rows_w, body, 0)
        pltpu.sync_copy(out_v, out_hbm.at[pl.ds(wid * out_rows_w, out_rows_w)])

    out_p = gather_kernel(pidx3, rem3, table_p)
    return out_p.reshape(B, D)
